# baseline (speedup 1.0000x reference)
import jax
import jax.numpy as jnp
from jax.experimental import pallas as pl
from jax.experimental.pallas import tpu as pltpu

_f32 = jnp.float32
_bf16 = jnp.bfloat16


def _make_upconv_kernel(H, W, TH):
    def body(ll_ref, lh_ref, hl_ref, hh_ref, wp_ref, b_ref, y_ref, stats_ref):
        Cin = ll_ref.shape[1]
        Cout = wp_ref.shape[4]
        M = TH * W
        SL = (TH + 2) * W

        r = pl.program_id(1)
        nrb = pl.num_programs(1)
        r0 = pl.multiple_of(r * TH, TH)

        def pair_cl(refa, refb, start, n, relu):
            va = refa[0, :, pl.ds(start, n), :].reshape(Cin, n * W)
            if relu:
                va = jnp.maximum(va, 0.0)
            vb = refb[0, :, pl.ds(start, n), :].reshape(Cin, n * W)
            return jnp.transpose(jnp.concatenate([va, vb], axis=0), (1, 0))

        tmask = (r > 0).astype(_f32)
        bmask = (r < nrb - 1).astype(_f32)
        top = jnp.maximum(r0 - 1, 0)
        bot = jnp.minimum(r0 + TH, H - 1)

        def slab(refa, refb, relu):
            t = pair_cl(refa, refb, top, 1, relu) * tmask
            m = pair_cl(refa, refb, r0, TH, relu)
            b = pair_cl(refa, refb, bot, 1, relu) * bmask
            return jnp.concatenate([t, m, b], axis=0)

        t1 = slab(ll_ref, lh_ref, True)
        t2 = slab(hl_ref, hh_ref, False)

        lane = jax.lax.broadcasted_iota(jnp.int32, (SL, 2 * Cin), 1)
        sgn = jnp.where(lane < Cin, -1.0, 1.0).astype(_f32)

        def rot(x):
            return jnp.concatenate([x[:, Cin:], x[:, :Cin]], axis=1)

        u = t1 - t2
        v = t1 + t2
        p0 = (0.5 * (u + sgn * rot(u))).astype(_bf16)
        p1 = (0.5 * (v + sgn * rot(v))).astype(_bf16)

        col = jax.lax.broadcasted_iota(jnp.int32, (SL, 2 * Cin), 0) % W
        ml = ((lane < Cin) & (col != W - 1)).astype(_bf16)
        mr = ((lane >= Cin) & (col != 0)).astype(_bf16)
        zrow = jnp.zeros((1, 2 * Cin), _bf16)

        def shifted(x):
            sp = jnp.concatenate([x[1:], zrow], axis=0)
            sn = jnp.concatenate([zrow, x[:-1]], axis=0)
            return sp * ml + sn * mr

        ps0 = shifted(p0)
        ps1 = shifted(p1)
        pb = (p0, p1)
        psb = (ps0, ps1)

        accs = []
        for p in (0, 1):
            for q in (0, 1):
                acc = jnp.broadcast_to(b_ref[...], (M, Cout))
                for dy in (-1, 0, 1):
                    s = (p + dy) & 1
                    beg = ((p + dy) >> 1) * W + W
                    acc = acc + jnp.dot(pb[s][beg:beg + M],
                                        wp_ref[q, dy + 1, 0],
                                        preferred_element_type=_f32)
                    acc = acc + jnp.dot(psb[s][beg:beg + M],
                                        wp_ref[q, dy + 1, 1],
                                        preferred_element_type=_f32)
                accs.append(acc.reshape(TH, W, Cout))

        even = jnp.stack([accs[0], accs[1]], axis=2)
        odd = jnp.stack([accs[2], accs[3]], axis=2)
        full = jnp.stack([even, odd], axis=1)
        fullb = full.reshape(4 * M, Cout).astype(_bf16)
        y_ref[0] = fullb

        ones = jnp.ones((1, 4 * M), _bf16)
        stats_ref[0, 0, 0:1, :] = jnp.dot(ones, fullb,
                                          preferred_element_type=_f32)
        stats_ref[0, 0, 1:2, :] = jnp.dot(ones, fullb * fullb,
                                          preferred_element_type=_f32)

    return body


def _make_bn_kernel(RG, W2, cnt, eps):
    def body(y_ref, stats_ref, gamma_ref, beta_ref, o_ref):
        s = jnp.sum(stats_ref[...].reshape(-1, 2, stats_ref.shape[-1]), axis=0)
        mean = s[0:1] / cnt
        var = jnp.maximum(s[1:2] / cnt - mean * mean, 0.0)
        scale = gamma_ref[...] * jax.lax.rsqrt(var + eps)
        shift = beta_ref[...] - mean * scale
        y = y_ref[0].astype(_f32) * scale + shift
        t = jnp.transpose(y, (1, 0))
        o_ref[0] = t.reshape(t.shape[0], RG, W2)

    return body


def kernel(LL, LH, HL, HH, conv_w, conv_b, bn_gamma, bn_beta, *, eps=1e-5):
    N, Cin, H, W = LL.shape
    Cout = conv_w.shape[0]
    OHW = 4 * H * W
    H2, W2 = 2 * H, 2 * W
    TH = 8 if H % 8 == 0 else H
    R = H // TH

    wt = jnp.transpose(conv_w, (2, 3, 1, 0))
    z = jnp.zeros((3, Cin, Cout), conv_w.dtype)
    wp = jnp.stack([
        jnp.stack([jnp.concatenate([wt[:, 1], wt[:, 2]], axis=1),
                   jnp.concatenate([z, wt[:, 0]], axis=1)], axis=1),
        jnp.stack([jnp.concatenate([wt[:, 0], wt[:, 1]], axis=1),
                   jnp.concatenate([wt[:, 2], z], axis=1)], axis=1),
    ], axis=0).astype(_bf16)
    b2 = conv_b.reshape(1, Cout).astype(_f32)

    band_spec = pl.BlockSpec((1, Cin, H, W), lambda n, r: (n, 0, 0, 0))

    y, stats = pl.pallas_call(
        _make_upconv_kernel(H, W, TH),
        out_shape=(jax.ShapeDtypeStruct((N, OHW, Cout), _bf16),
                   jax.ShapeDtypeStruct((N, R, 2, Cout), _f32)),
        grid_spec=pltpu.PrefetchScalarGridSpec(
            num_scalar_prefetch=0,
            grid=(N, R),
            in_specs=[band_spec, band_spec, band_spec, band_spec,
                      pl.BlockSpec((2, 3, 2, 2 * Cin, Cout),
                                   lambda n, r: (0, 0, 0, 0, 0)),
                      pl.BlockSpec((1, Cout), lambda n, r: (0, 0))],
            out_specs=(pl.BlockSpec((1, 4 * TH * W, Cout),
                                    lambda n, r: (n, r, 0)),
                       pl.BlockSpec((1, 1, 2, Cout),
                                    lambda n, r: (n, r, 0, 0)))),
        compiler_params=pltpu.CompilerParams(
            dimension_semantics=("parallel", "parallel")),
    )(LL, LH, HL, HH, wp, b2)

    gam = bn_gamma.reshape(1, Cout).astype(_f32)
    bet = bn_beta.reshape(1, Cout).astype(_f32)
    RG = 16 if H2 % 16 == 0 else 1
    R2 = H2 // RG
    L2 = RG * W2
    y_bn = pl.pallas_call(
        _make_bn_kernel(RG, W2, float(N * OHW), eps),
        out_shape=jax.ShapeDtypeStruct((N, Cout, H2, W2), _f32),
        grid_spec=pltpu.PrefetchScalarGridSpec(
            num_scalar_prefetch=0,
            grid=(N, R2),
            in_specs=[pl.BlockSpec((1, L2, Cout), lambda n, r: (n, r, 0)),
                      pl.BlockSpec((N, R, 2, Cout), lambda n, r: (0, 0, 0, 0)),
                      pl.BlockSpec((1, Cout), lambda n, r: (0, 0)),
                      pl.BlockSpec((1, Cout), lambda n, r: (0, 0))],
            out_specs=pl.BlockSpec((1, Cout, RG, W2), lambda n, r: (n, 0, r, 0))),
        compiler_params=pltpu.CompilerParams(
            dimension_semantics=("parallel", "parallel")),
    )(y, stats, gam, bet)

    return y_bn


# speedup vs baseline: 1.1772x; 1.1772x over previous
import jax
import jax.numpy as jnp
from jax.experimental import pallas as pl
from jax.experimental.pallas import tpu as pltpu

_f32 = jnp.float32
_bf16 = jnp.bfloat16


def _make_upconv_kernel(H, W, TH):
    def body(ll_ref, lh_ref, hl_ref, hh_ref, wp_ref, b_ref, y_ref, stats_ref):
        Cin = ll_ref.shape[1]
        Cout = wp_ref.shape[4]
        M = TH * W
        SL = (TH + 2) * W

        r = pl.program_id(1)
        nrb = pl.num_programs(1)
        r0 = pl.multiple_of(r * TH, TH)

        def pair_cl(refa, refb, start, n, relu):
            va = refa[0, :, pl.ds(start, n), :].reshape(Cin, n * W)
            if relu:
                va = jnp.maximum(va, 0.0)
            vb = refb[0, :, pl.ds(start, n), :].reshape(Cin, n * W)
            return jnp.transpose(jnp.concatenate([va, vb], axis=0), (1, 0))

        tmask = (r > 0).astype(_f32)
        bmask = (r < nrb - 1).astype(_f32)
        top = jnp.maximum(r0 - 1, 0)
        bot = jnp.minimum(r0 + TH, H - 1)

        def slab(refa, refb, relu):
            t = pair_cl(refa, refb, top, 1, relu) * tmask
            m = pair_cl(refa, refb, r0, TH, relu)
            b = pair_cl(refa, refb, bot, 1, relu) * bmask
            return jnp.concatenate([t, m, b], axis=0)

        t1 = slab(ll_ref, lh_ref, True)
        t2 = slab(hl_ref, hh_ref, False)

        lane = jax.lax.broadcasted_iota(jnp.int32, (SL, 2 * Cin), 1)
        sgn = jnp.where(lane < Cin, -1.0, 1.0).astype(_f32)

        def rot(x):
            return jnp.concatenate([x[:, Cin:], x[:, :Cin]], axis=1)

        u = t1 - t2
        v = t1 + t2
        p0 = (0.5 * (u + sgn * rot(u))).astype(_bf16)
        p1 = (0.5 * (v + sgn * rot(v))).astype(_bf16)

        col = jax.lax.broadcasted_iota(jnp.int32, (SL, 2 * Cin), 0) % W
        ml = ((lane < Cin) & (col != W - 1)).astype(_bf16)
        mr = ((lane >= Cin) & (col != 0)).astype(_bf16)
        zrow = jnp.zeros((1, 2 * Cin), _bf16)

        def shifted(x):
            sp = jnp.concatenate([x[1:], zrow], axis=0)
            sn = jnp.concatenate([zrow, x[:-1]], axis=0)
            return sp * ml + sn * mr

        ps0 = shifted(p0)
        ps1 = shifted(p1)
        pb = (p0, p1)
        psb = (ps0, ps1)

        accs = []
        for p in (0, 1):
            for q in (0, 1):
                acc = jnp.broadcast_to(b_ref[...], (M, Cout))
                for dy in (-1, 0, 1):
                    s = (p + dy) & 1
                    beg = ((p + dy) >> 1) * W + W
                    acc = acc + jnp.dot(pb[s][beg:beg + M],
                                        wp_ref[q, dy + 1, 0],
                                        preferred_element_type=_f32)
                    acc = acc + jnp.dot(psb[s][beg:beg + M],
                                        wp_ref[q, dy + 1, 1],
                                        preferred_element_type=_f32)
                accs.append(acc.reshape(TH, W, Cout))

        even = jnp.stack([accs[0], accs[1]], axis=2)
        odd = jnp.stack([accs[2], accs[3]], axis=2)
        full = jnp.stack([even, odd], axis=1)
        fullb = full.reshape(4 * M, Cout).astype(_bf16)
        y_ref[0] = fullb

        ones = jnp.ones((1, 4 * M), _bf16)
        stats_ref[0, 0, 0:1, :] = jnp.dot(ones, fullb,
                                          preferred_element_type=_f32)
        stats_ref[0, 0, 1:2, :] = jnp.dot(ones, fullb * fullb,
                                          preferred_element_type=_f32)

    return body


def _make_bn_kernel(RG, W2, cnt, eps):
    def body(y_ref, stats_ref, gamma_ref, beta_ref, o_ref):
        s = jnp.sum(stats_ref[...].reshape(-1, 2, stats_ref.shape[-1]), axis=0)
        mean = s[0:1] / cnt
        var = jnp.maximum(s[1:2] / cnt - mean * mean, 0.0)
        scale = gamma_ref[...] * jax.lax.rsqrt(var + eps)
        shift = beta_ref[...] - mean * scale
        y = y_ref[0].astype(_f32) * scale + shift
        t = jnp.transpose(y, (1, 0))
        o_ref[0] = t.reshape(t.shape[0], RG, W2)

    return body


def kernel(LL, LH, HL, HH, conv_w, conv_b, bn_gamma, bn_beta, *, eps=1e-5):
    N, Cin, H, W = LL.shape
    Cout = conv_w.shape[0]
    OHW = 4 * H * W
    H2, W2 = 2 * H, 2 * W
    TH = 32 if H % 32 == 0 else H
    R = H // TH

    wt = jnp.transpose(conv_w, (2, 3, 1, 0))
    z = jnp.zeros((3, Cin, Cout), conv_w.dtype)
    wp = jnp.stack([
        jnp.stack([jnp.concatenate([wt[:, 1], wt[:, 2]], axis=1),
                   jnp.concatenate([z, wt[:, 0]], axis=1)], axis=1),
        jnp.stack([jnp.concatenate([wt[:, 0], wt[:, 1]], axis=1),
                   jnp.concatenate([wt[:, 2], z], axis=1)], axis=1),
    ], axis=0).astype(_bf16)
    b2 = conv_b.reshape(1, Cout).astype(_f32)

    band_spec = pl.BlockSpec((1, Cin, H, W), lambda n, r: (n, 0, 0, 0))

    y, stats = pl.pallas_call(
        _make_upconv_kernel(H, W, TH),
        out_shape=(jax.ShapeDtypeStruct((N, OHW, Cout), _bf16),
                   jax.ShapeDtypeStruct((N, R, 2, Cout), _f32)),
        grid_spec=pltpu.PrefetchScalarGridSpec(
            num_scalar_prefetch=0,
            grid=(N, R),
            in_specs=[band_spec, band_spec, band_spec, band_spec,
                      pl.BlockSpec((2, 3, 2, 2 * Cin, Cout),
                                   lambda n, r: (0, 0, 0, 0, 0)),
                      pl.BlockSpec((1, Cout), lambda n, r: (0, 0))],
            out_specs=(pl.BlockSpec((1, 4 * TH * W, Cout),
                                    lambda n, r: (n, r, 0)),
                       pl.BlockSpec((1, 1, 2, Cout),
                                    lambda n, r: (n, r, 0, 0)))),
        compiler_params=pltpu.CompilerParams(
            dimension_semantics=("parallel", "parallel")),
    )(LL, LH, HL, HH, wp, b2)

    gam = bn_gamma.reshape(1, Cout).astype(_f32)
    bet = bn_beta.reshape(1, Cout).astype(_f32)
    RG = 16 if H2 % 16 == 0 else 1
    R2 = H2 // RG
    L2 = RG * W2
    y_bn = pl.pallas_call(
        _make_bn_kernel(RG, W2, float(N * OHW), eps),
        out_shape=jax.ShapeDtypeStruct((N, Cout, H2, W2), _f32),
        grid_spec=pltpu.PrefetchScalarGridSpec(
            num_scalar_prefetch=0,
            grid=(N, R2),
            in_specs=[pl.BlockSpec((1, L2, Cout), lambda n, r: (n, r, 0)),
                      pl.BlockSpec((N, R, 2, Cout), lambda n, r: (0, 0, 0, 0)),
                      pl.BlockSpec((1, Cout), lambda n, r: (0, 0)),
                      pl.BlockSpec((1, Cout), lambda n, r: (0, 0))],
            out_specs=pl.BlockSpec((1, Cout, RG, W2), lambda n, r: (n, 0, r, 0))),
        compiler_params=pltpu.CompilerParams(
            dimension_semantics=("parallel", "parallel")),
    )(y, stats, gam, bet)

    return y_bn


# speedup vs baseline: 1.2710x; 1.0797x over previous
import jax
import jax.numpy as jnp
from jax.experimental import pallas as pl
from jax.experimental.pallas import tpu as pltpu

_f32 = jnp.float32
_bf16 = jnp.bfloat16


def _make_upconv_kernel(H, W, TH):
    def body(ll_ref, lh_ref, hl_ref, hh_ref, wp_ref, b_ref, y_ref, stats_ref):
        Cin = ll_ref.shape[1]
        Cout = wp_ref.shape[4]
        M = TH * W
        SL = (TH + 2) * W

        r = pl.program_id(1)
        nrb = pl.num_programs(1)
        r0 = pl.multiple_of(r * TH, TH)

        def pair_cl(refa, refb, start, n, relu):
            va = refa[0, :, pl.ds(start, n), :].reshape(Cin, n * W)
            if relu:
                va = jnp.maximum(va, 0.0)
            vb = refb[0, :, pl.ds(start, n), :].reshape(Cin, n * W)
            return jnp.transpose(jnp.concatenate([va, vb], axis=0), (1, 0))

        tmask = (r > 0).astype(_f32)
        bmask = (r < nrb - 1).astype(_f32)
        top = jnp.maximum(r0 - 1, 0)
        bot = jnp.minimum(r0 + TH, H - 1)

        def slab(refa, refb, relu):
            t = pair_cl(refa, refb, top, 1, relu) * tmask
            m = pair_cl(refa, refb, r0, TH, relu)
            b = pair_cl(refa, refb, bot, 1, relu) * bmask
            return jnp.concatenate([t, m, b], axis=0)

        t1 = slab(ll_ref, lh_ref, True)
        t2 = slab(hl_ref, hh_ref, False)

        lane = jax.lax.broadcasted_iota(jnp.int32, (SL, 2 * Cin), 1)
        sgn = jnp.where(lane < Cin, -1.0, 1.0).astype(_f32)

        def rot(x):
            return jnp.concatenate([x[:, Cin:], x[:, :Cin]], axis=1)

        u = t1 - t2
        v = t1 + t2
        p0 = (0.5 * (u + sgn * rot(u))).astype(_bf16)
        p1 = (0.5 * (v + sgn * rot(v))).astype(_bf16)

        col = jax.lax.broadcasted_iota(jnp.int32, (SL, 2 * Cin), 0) % W
        ml = ((lane < Cin) & (col != W - 1)).astype(_bf16)
        mr = ((lane >= Cin) & (col != 0)).astype(_bf16)
        zrow = jnp.zeros((1, 2 * Cin), _bf16)

        def shifted(x):
            sp = jnp.concatenate([x[1:], zrow], axis=0)
            sn = jnp.concatenate([zrow, x[:-1]], axis=0)
            return sp * ml + sn * mr

        ps0 = shifted(p0)
        ps1 = shifted(p1)
        pb = (p0, p1)
        psb = (ps0, ps1)

        accs = []
        for p in (0, 1):
            for q in (0, 1):
                acc = jnp.broadcast_to(b_ref[...], (M, Cout))
                for dy in (-1, 0, 1):
                    s = (p + dy) & 1
                    beg = ((p + dy) >> 1) * W + W
                    acc = acc + jnp.dot(pb[s][beg:beg + M],
                                        wp_ref[q, dy + 1, 0],
                                        preferred_element_type=_f32)
                    acc = acc + jnp.dot(psb[s][beg:beg + M],
                                        wp_ref[q, dy + 1, 1],
                                        preferred_element_type=_f32)
                accs.append(acc.reshape(TH, W, Cout))

        even = jnp.stack([accs[0], accs[1]], axis=2)
        odd = jnp.stack([accs[2], accs[3]], axis=2)
        full = jnp.stack([even, odd], axis=1)
        fullb = full.reshape(4 * M, Cout).astype(_bf16)
        y_ref[0] = fullb

        ones = jnp.ones((1, 4 * M), _bf16)
        stats_ref[0, 0, 0:1, :] = jnp.dot(ones, fullb,
                                          preferred_element_type=_f32)
        stats_ref[0, 0, 1:2, :] = jnp.dot(ones, fullb * fullb,
                                          preferred_element_type=_f32)

    return body


def _make_bn_kernel(RG, W2, cnt, eps):
    def body(y_ref, stats_ref, gamma_ref, beta_ref, o_ref):
        s = jnp.sum(stats_ref[...].reshape(-1, 2, stats_ref.shape[-1]), axis=0)
        mean = s[0:1] / cnt
        var = jnp.maximum(s[1:2] / cnt - mean * mean, 0.0)
        scale = gamma_ref[...] * jax.lax.rsqrt(var + eps)
        shift = beta_ref[...] - mean * scale
        y = y_ref[0].astype(_f32) * scale + shift
        t = jnp.transpose(y, (1, 0))
        o_ref[0] = t.reshape(t.shape[0], RG, W2)

    return body


def kernel(LL, LH, HL, HH, conv_w, conv_b, bn_gamma, bn_beta, *, eps=1e-5):
    N, Cin, H, W = LL.shape
    Cout = conv_w.shape[0]
    OHW = 4 * H * W
    H2, W2 = 2 * H, 2 * W
    TH = 32 if H % 32 == 0 else H
    R = H // TH

    wt = jnp.transpose(conv_w, (2, 3, 1, 0))
    z = jnp.zeros((3, Cin, Cout), conv_w.dtype)
    wp = jnp.stack([
        jnp.stack([jnp.concatenate([wt[:, 1], wt[:, 2]], axis=1),
                   jnp.concatenate([z, wt[:, 0]], axis=1)], axis=1),
        jnp.stack([jnp.concatenate([wt[:, 0], wt[:, 1]], axis=1),
                   jnp.concatenate([wt[:, 2], z], axis=1)], axis=1),
    ], axis=0).astype(_bf16)
    b2 = conv_b.reshape(1, Cout).astype(_f32)

    band_spec = pl.BlockSpec((1, Cin, H, W), lambda n, r: (n, 0, 0, 0))

    y, stats = pl.pallas_call(
        _make_upconv_kernel(H, W, TH),
        out_shape=(jax.ShapeDtypeStruct((N, OHW, Cout), _bf16),
                   jax.ShapeDtypeStruct((N, R, 2, Cout), _f32)),
        grid_spec=pltpu.PrefetchScalarGridSpec(
            num_scalar_prefetch=0,
            grid=(N, R),
            in_specs=[band_spec, band_spec, band_spec, band_spec,
                      pl.BlockSpec((2, 3, 2, 2 * Cin, Cout),
                                   lambda n, r: (0, 0, 0, 0, 0)),
                      pl.BlockSpec((1, Cout), lambda n, r: (0, 0))],
            out_specs=(pl.BlockSpec((1, 4 * TH * W, Cout),
                                    lambda n, r: (n, r, 0)),
                       pl.BlockSpec((1, 1, 2, Cout),
                                    lambda n, r: (n, r, 0, 0)))),
        compiler_params=pltpu.CompilerParams(
            dimension_semantics=("parallel", "parallel")),
    )(LL, LH, HL, HH, wp, b2)

    gam = bn_gamma.reshape(1, Cout).astype(_f32)
    bet = bn_beta.reshape(1, Cout).astype(_f32)
    RG = 32 if H2 % 32 == 0 else 1
    R2 = H2 // RG
    L2 = RG * W2
    y_bn = pl.pallas_call(
        _make_bn_kernel(RG, W2, float(N * OHW), eps),
        out_shape=jax.ShapeDtypeStruct((N, Cout, H2, W2), _f32),
        grid_spec=pltpu.PrefetchScalarGridSpec(
            num_scalar_prefetch=0,
            grid=(N, R2),
            in_specs=[pl.BlockSpec((1, L2, Cout), lambda n, r: (n, r, 0)),
                      pl.BlockSpec((N, R, 2, Cout), lambda n, r: (0, 0, 0, 0)),
                      pl.BlockSpec((1, Cout), lambda n, r: (0, 0)),
                      pl.BlockSpec((1, Cout), lambda n, r: (0, 0))],
            out_specs=pl.BlockSpec((1, Cout, RG, W2), lambda n, r: (n, 0, r, 0))),
        compiler_params=pltpu.CompilerParams(
            dimension_semantics=("parallel", "parallel")),
    )(y, stats, gam, bet)

    return y_bn


# speedup vs baseline: 1.3345x; 1.0499x over previous
import jax
import jax.numpy as jnp
from jax.experimental import pallas as pl
from jax.experimental.pallas import tpu as pltpu

_f32 = jnp.float32
_bf16 = jnp.bfloat16


def _make_upconv_kernel(H, W, TH):
    def body(ll_ref, lh_ref, hl_ref, hh_ref, wp_ref, b_ref, y_ref, stats_ref):
        Cin = ll_ref.shape[1]
        Cout = wp_ref.shape[4]
        M = TH * W
        SL = (TH + 2) * W

        r = pl.program_id(1)
        nrb = pl.num_programs(1)
        r0 = pl.multiple_of(r * TH, TH)

        def pair_cl(refa, refb, start, n, relu):
            va = refa[0, :, pl.ds(start, n), :].reshape(Cin, n * W)
            if relu:
                va = jnp.maximum(va, 0.0)
            vb = refb[0, :, pl.ds(start, n), :].reshape(Cin, n * W)
            return jnp.transpose(jnp.concatenate([va, vb], axis=0), (1, 0))

        tmask = (r > 0).astype(_f32)
        bmask = (r < nrb - 1).astype(_f32)
        top = jnp.maximum(r0 - 1, 0)
        bot = jnp.minimum(r0 + TH, H - 1)

        def slab(refa, refb, relu):
            t = pair_cl(refa, refb, top, 1, relu) * tmask
            m = pair_cl(refa, refb, r0, TH, relu)
            b = pair_cl(refa, refb, bot, 1, relu) * bmask
            return jnp.concatenate([t, m, b], axis=0)

        t1 = slab(ll_ref, lh_ref, True)
        t2 = slab(hl_ref, hh_ref, False)

        lane = jax.lax.broadcasted_iota(jnp.int32, (SL, 2 * Cin), 1)
        sgn = jnp.where(lane < Cin, -1.0, 1.0).astype(_f32)

        def rot(x):
            return jnp.concatenate([x[:, Cin:], x[:, :Cin]], axis=1)

        u = t1 - t2
        v = t1 + t2
        p0 = (0.5 * (u + sgn * rot(u))).astype(_bf16)
        p1 = (0.5 * (v + sgn * rot(v))).astype(_bf16)

        col = jax.lax.broadcasted_iota(jnp.int32, (SL, 2 * Cin), 0) % W
        ml = ((lane < Cin) & (col != W - 1)).astype(_bf16)
        mr = ((lane >= Cin) & (col != 0)).astype(_bf16)
        zrow = jnp.zeros((1, 2 * Cin), _bf16)

        def shifted(x):
            sp = jnp.concatenate([x[1:], zrow], axis=0)
            sn = jnp.concatenate([zrow, x[:-1]], axis=0)
            return sp * ml + sn * mr

        ps0 = shifted(p0)
        ps1 = shifted(p1)
        pb = (p0, p1)
        psb = (ps0, ps1)

        accs = []
        for p in (0, 1):
            for q in (0, 1):
                acc = jnp.broadcast_to(b_ref[...], (M, Cout))
                for dy in (-1, 0, 1):
                    s = (p + dy) & 1
                    beg = ((p + dy) >> 1) * W + W
                    acc = acc + jnp.dot(pb[s][beg:beg + M],
                                        wp_ref[q, dy + 1, 0],
                                        preferred_element_type=_f32)
                    acc = acc + jnp.dot(psb[s][beg:beg + M],
                                        wp_ref[q, dy + 1, 1],
                                        preferred_element_type=_f32)
                accs.append(acc.reshape(TH, W, Cout))

        even = jnp.stack([accs[0], accs[1]], axis=2)
        odd = jnp.stack([accs[2], accs[3]], axis=2)
        full = jnp.stack([even, odd], axis=1)
        fullb = full.reshape(4 * M, Cout).astype(_bf16)
        y_ref[0] = fullb

        ones = jnp.ones((1, 4 * M), _bf16)
        stats_ref[0, 0, 0:1, :] = jnp.dot(ones, fullb,
                                          preferred_element_type=_f32)
        stats_ref[0, 0, 1:2, :] = jnp.dot(ones, fullb * fullb,
                                          preferred_element_type=_f32)

    return body


def _make_bn_kernel(RG, W2, cnt, eps):
    def body(y_ref, stats_ref, gamma_ref, beta_ref, o_ref):
        s = jnp.sum(stats_ref[...].reshape(-1, 2, stats_ref.shape[-1]), axis=0)
        mean = s[0:1] / cnt
        var = jnp.maximum(s[1:2] / cnt - mean * mean, 0.0)
        scale = gamma_ref[...] * jax.lax.rsqrt(var + eps)
        shift = beta_ref[...] - mean * scale
        y = y_ref[0].astype(_f32) * scale + shift
        t = jnp.transpose(y, (1, 0))
        o_ref[0] = t.reshape(t.shape[0], RG, W2)

    return body


def kernel(LL, LH, HL, HH, conv_w, conv_b, bn_gamma, bn_beta, *, eps=1e-5):
    N, Cin, H, W = LL.shape
    Cout = conv_w.shape[0]
    OHW = 4 * H * W
    H2, W2 = 2 * H, 2 * W
    TH = 32 if H % 32 == 0 else H
    R = H // TH

    wt = jnp.transpose(conv_w, (2, 3, 1, 0))
    z = jnp.zeros((3, Cin, Cout), conv_w.dtype)
    wp = jnp.stack([
        jnp.stack([jnp.concatenate([wt[:, 1], wt[:, 2]], axis=1),
                   jnp.concatenate([z, wt[:, 0]], axis=1)], axis=1),
        jnp.stack([jnp.concatenate([wt[:, 0], wt[:, 1]], axis=1),
                   jnp.concatenate([wt[:, 2], z], axis=1)], axis=1),
    ], axis=0).astype(_bf16)
    b2 = conv_b.reshape(1, Cout).astype(_f32)

    band_spec = pl.BlockSpec((1, Cin, H, W), lambda n, r: (n, 0, 0, 0))

    y, stats = pl.pallas_call(
        _make_upconv_kernel(H, W, TH),
        out_shape=(jax.ShapeDtypeStruct((N, OHW, Cout), _bf16),
                   jax.ShapeDtypeStruct((N, R, 2, Cout), _f32)),
        grid_spec=pltpu.PrefetchScalarGridSpec(
            num_scalar_prefetch=0,
            grid=(N, R),
            in_specs=[band_spec, band_spec, band_spec, band_spec,
                      pl.BlockSpec((2, 3, 2, 2 * Cin, Cout),
                                   lambda n, r: (0, 0, 0, 0, 0)),
                      pl.BlockSpec((1, Cout), lambda n, r: (0, 0))],
            out_specs=(pl.BlockSpec((1, 4 * TH * W, Cout),
                                    lambda n, r: (n, r, 0)),
                       pl.BlockSpec((1, 1, 2, Cout),
                                    lambda n, r: (n, r, 0, 0)))),
        compiler_params=pltpu.CompilerParams(
            dimension_semantics=("parallel", "parallel")),
    )(LL, LH, HL, HH, wp, b2)

    gam = bn_gamma.reshape(1, Cout).astype(_f32)
    bet = bn_beta.reshape(1, Cout).astype(_f32)
    RG = 64 if H2 % 64 == 0 else 1
    R2 = H2 // RG
    L2 = RG * W2
    y_bn = pl.pallas_call(
        _make_bn_kernel(RG, W2, float(N * OHW), eps),
        out_shape=jax.ShapeDtypeStruct((N, Cout, H2, W2), _f32),
        grid_spec=pltpu.PrefetchScalarGridSpec(
            num_scalar_prefetch=0,
            grid=(N, R2),
            in_specs=[pl.BlockSpec((1, L2, Cout), lambda n, r: (n, r, 0)),
                      pl.BlockSpec((N, R, 2, Cout), lambda n, r: (0, 0, 0, 0)),
                      pl.BlockSpec((1, Cout), lambda n, r: (0, 0)),
                      pl.BlockSpec((1, Cout), lambda n, r: (0, 0))],
            out_specs=pl.BlockSpec((1, Cout, RG, W2), lambda n, r: (n, 0, r, 0))),
        compiler_params=pltpu.CompilerParams(
            dimension_semantics=("parallel", "parallel")),
    )(y, stats, gam, bet)

    return y_bn


# speedup vs baseline: 1.3694x; 1.0261x over previous
import jax
import jax.numpy as jnp
from jax.experimental import pallas as pl
from jax.experimental.pallas import tpu as pltpu

_f32 = jnp.float32
_bf16 = jnp.bfloat16


def _make_upconv_kernel(H, W, TH):
    def body(ll_ref, lh_ref, hl_ref, hh_ref, wp_ref, b_ref, y_ref, stats_ref):
        Cin = ll_ref.shape[1]
        Cout = wp_ref.shape[4]
        M = TH * W
        SL = (TH + 2) * W

        r = pl.program_id(1)
        nrb = pl.num_programs(1)
        r0 = pl.multiple_of(r * TH, TH)

        def pair_cl(refa, refb, start, n, relu):
            va = refa[0, :, pl.ds(start, n), :].reshape(Cin, n * W)
            if relu:
                va = jnp.maximum(va, 0.0)
            vb = refb[0, :, pl.ds(start, n), :].reshape(Cin, n * W)
            return jnp.transpose(jnp.concatenate([va, vb], axis=0), (1, 0))

        tmask = (r > 0).astype(_f32)
        bmask = (r < nrb - 1).astype(_f32)
        top = jnp.maximum(r0 - 1, 0)
        bot = jnp.minimum(r0 + TH, H - 1)

        def slab(refa, refb, relu):
            t = pair_cl(refa, refb, top, 1, relu) * tmask
            m = pair_cl(refa, refb, r0, TH, relu)
            b = pair_cl(refa, refb, bot, 1, relu) * bmask
            return jnp.concatenate([t, m, b], axis=0)

        t1 = slab(ll_ref, lh_ref, True)
        t2 = slab(hl_ref, hh_ref, False)

        lane = jax.lax.broadcasted_iota(jnp.int32, (SL, 2 * Cin), 1)
        sgn = jnp.where(lane < Cin, -1.0, 1.0).astype(_f32)

        def rot(x):
            return jnp.concatenate([x[:, Cin:], x[:, :Cin]], axis=1)

        u = t1 - t2
        v = t1 + t2
        p0 = (0.5 * (u + sgn * rot(u))).astype(_bf16)
        p1 = (0.5 * (v + sgn * rot(v))).astype(_bf16)

        col = jax.lax.broadcasted_iota(jnp.int32, (SL, 2 * Cin), 0) % W
        ml = ((lane < Cin) & (col != W - 1)).astype(_bf16)
        mr = ((lane >= Cin) & (col != 0)).astype(_bf16)
        zrow = jnp.zeros((1, 2 * Cin), _bf16)

        def shifted(x):
            sp = jnp.concatenate([x[1:], zrow], axis=0)
            sn = jnp.concatenate([zrow, x[:-1]], axis=0)
            return sp * ml + sn * mr

        ps0 = shifted(p0)
        ps1 = shifted(p1)
        pb = (p0, p1)
        psb = (ps0, ps1)

        accs = []
        for p in (0, 1):
            for q in (0, 1):
                acc = jnp.broadcast_to(b_ref[...], (M, Cout))
                for dy in (-1, 0, 1):
                    s = (p + dy) & 1
                    beg = ((p + dy) >> 1) * W + W
                    acc = acc + jnp.dot(pb[s][beg:beg + M],
                                        wp_ref[q, dy + 1, 0],
                                        preferred_element_type=_f32)
                    acc = acc + jnp.dot(psb[s][beg:beg + M],
                                        wp_ref[q, dy + 1, 1],
                                        preferred_element_type=_f32)
                accs.append(acc.reshape(TH, W, Cout))

        even = jnp.stack([accs[0], accs[1]], axis=2)
        odd = jnp.stack([accs[2], accs[3]], axis=2)
        full = jnp.stack([even, odd], axis=1)
        fullb = full.reshape(4 * M, Cout).astype(_bf16)
        y_ref[0] = fullb

        ones = jnp.ones((1, 4 * M), _bf16)
        stats_ref[0, 0, 0:1, :] = jnp.dot(ones, fullb,
                                          preferred_element_type=_f32)
        stats_ref[0, 0, 1:2, :] = jnp.dot(ones, fullb * fullb,
                                          preferred_element_type=_f32)

    return body


def _make_bn_kernel(RG, W2, cnt, eps):
    def body(y_ref, stats_ref, gamma_ref, beta_ref, o_ref):
        s = jnp.sum(stats_ref[...].reshape(-1, 2, stats_ref.shape[-1]), axis=0)
        mean = s[0:1] / cnt
        var = jnp.maximum(s[1:2] / cnt - mean * mean, 0.0)
        scale = gamma_ref[...] * jax.lax.rsqrt(var + eps)
        shift = beta_ref[...] - mean * scale
        y = y_ref[0].astype(_f32) * scale + shift
        t = jnp.transpose(y, (1, 0))
        o_ref[0] = t.reshape(t.shape[0], RG, W2)

    return body


def kernel(LL, LH, HL, HH, conv_w, conv_b, bn_gamma, bn_beta, *, eps=1e-5):
    N, Cin, H, W = LL.shape
    Cout = conv_w.shape[0]
    OHW = 4 * H * W
    H2, W2 = 2 * H, 2 * W
    TH = 32 if H % 32 == 0 else H
    R = H // TH

    wt = jnp.transpose(conv_w, (2, 3, 1, 0))
    z = jnp.zeros((3, Cin, Cout), conv_w.dtype)
    wp = jnp.stack([
        jnp.stack([jnp.concatenate([wt[:, 1], wt[:, 2]], axis=1),
                   jnp.concatenate([z, wt[:, 0]], axis=1)], axis=1),
        jnp.stack([jnp.concatenate([wt[:, 0], wt[:, 1]], axis=1),
                   jnp.concatenate([wt[:, 2], z], axis=1)], axis=1),
    ], axis=0).astype(_bf16)
    b2 = conv_b.reshape(1, Cout).astype(_f32)

    band_spec = pl.BlockSpec((1, Cin, H, W), lambda n, r: (n, 0, 0, 0))

    y, stats = pl.pallas_call(
        _make_upconv_kernel(H, W, TH),
        out_shape=(jax.ShapeDtypeStruct((N, OHW, Cout), _bf16),
                   jax.ShapeDtypeStruct((N, R, 2, Cout), _f32)),
        grid_spec=pltpu.PrefetchScalarGridSpec(
            num_scalar_prefetch=0,
            grid=(N, R),
            in_specs=[band_spec, band_spec, band_spec, band_spec,
                      pl.BlockSpec((2, 3, 2, 2 * Cin, Cout),
                                   lambda n, r: (0, 0, 0, 0, 0)),
                      pl.BlockSpec((1, Cout), lambda n, r: (0, 0))],
            out_specs=(pl.BlockSpec((1, 4 * TH * W, Cout),
                                    lambda n, r: (n, r, 0)),
                       pl.BlockSpec((1, 1, 2, Cout),
                                    lambda n, r: (n, r, 0, 0)))),
        compiler_params=pltpu.CompilerParams(
            dimension_semantics=("parallel", "parallel")),
    )(LL, LH, HL, HH, wp, b2)

    gam = bn_gamma.reshape(1, Cout).astype(_f32)
    bet = bn_beta.reshape(1, Cout).astype(_f32)
    RG = 128 if H2 % 128 == 0 else 1
    R2 = H2 // RG
    L2 = RG * W2
    y_bn = pl.pallas_call(
        _make_bn_kernel(RG, W2, float(N * OHW), eps),
        out_shape=jax.ShapeDtypeStruct((N, Cout, H2, W2), _f32),
        grid_spec=pltpu.PrefetchScalarGridSpec(
            num_scalar_prefetch=0,
            grid=(N, R2),
            in_specs=[pl.BlockSpec((1, L2, Cout), lambda n, r: (n, r, 0)),
                      pl.BlockSpec((N, R, 2, Cout), lambda n, r: (0, 0, 0, 0)),
                      pl.BlockSpec((1, Cout), lambda n, r: (0, 0)),
                      pl.BlockSpec((1, Cout), lambda n, r: (0, 0))],
            out_specs=pl.BlockSpec((1, Cout, RG, W2), lambda n, r: (n, 0, r, 0))),
        compiler_params=pltpu.CompilerParams(
            dimension_semantics=("parallel", "parallel")),
    )(y, stats, gam, bet)

    return y_bn
